# baseline (speedup 1.0000x reference)
import functools

import jax
import jax.numpy as jnp
from jax import lax
from jax.experimental import pallas as pl
from jax.experimental.pallas import tpu as pltpu
from jax.experimental.pallas import tpu_sc as plsc

T, D = 32768, 2048
NC, NS = 2, 16
NW = NC * NS
ROWS_PER_W = T // NW
C = 16
NCHUNK = ROWS_PER_W // C


def _scale_rows(buf, s0b, s1b, g):
    svec = s0b[pl.ds(g * C, C)] + s1b[pl.ds(g * C, C)]
    for r in range(C):
        ssum = svec[r]

        @plsc.parallel_loop(0, D, step=16, unroll=8)
        def _vec(j):
            buf[r, pl.ds(j, 16)] = buf[r, pl.ds(j, 16)] * ssum


def kernel(hidden_states, expert_scales):
    s0 = expert_scales[:, 0]
    s1 = expert_scales[:, 1]
    mesh = plsc.VectorSubcoreMesh(core_axis_name="c", subcore_axis_name="s")

    @functools.partial(
        pl.kernel,
        out_type=jax.ShapeDtypeStruct((T, D), jnp.float32),
        mesh=mesh,
        scratch_types=[
            pltpu.VMEM((C, D), jnp.float32),
            pltpu.VMEM((ROWS_PER_W,), jnp.float32),
            pltpu.VMEM((ROWS_PER_W,), jnp.float32),
        ],
    )
    def run(h_hbm, s0_hbm, s1_hbm, out_hbm, buf, s0b, s1b):
        wid = lax.axis_index("s") * NC + lax.axis_index("c")
        base = wid * ROWS_PER_W
        pltpu.sync_copy(s0_hbm.at[pl.ds(base, ROWS_PER_W)], s0b)
        pltpu.sync_copy(s1_hbm.at[pl.ds(base, ROWS_PER_W)], s1b)

        @pl.loop(0, NCHUNK)
        def _chunk(g):
            r0 = base + g * C
            pltpu.sync_copy(h_hbm.at[pl.ds(r0, C)], buf)
            _scale_rows(buf, s0b, s1b, g)
            pltpu.sync_copy(buf, out_hbm.at[pl.ds(r0, C)])

    return run(hidden_states, s0, s1)


# speedup vs baseline: 1.5629x; 1.5629x over previous
import functools

import jax
import jax.numpy as jnp
from jax import lax
from jax.experimental import pallas as pl
from jax.experimental.pallas import tpu as pltpu
from jax.experimental.pallas import tpu_sc as plsc

T, D = 32768, 2048
NC, NS = 2, 16
NW = NC * NS
ROWS_PER_W = T // NW
C = 8
NCH = ROWS_PER_W // C
NGRP = NCH // 2


def kernel(hidden_states, expert_scales):
    s0 = expert_scales[:, 0]
    s1 = expert_scales[:, 1]
    mesh = plsc.VectorSubcoreMesh(core_axis_name="c", subcore_axis_name="s")

    @functools.partial(
        pl.kernel,
        out_type=jax.ShapeDtypeStruct((T, D), jnp.float32),
        mesh=mesh,
        scratch_types=[
            pltpu.VMEM((C, D), jnp.float32),
            pltpu.VMEM((C, D), jnp.float32),
            pltpu.VMEM((C, D), jnp.float32),
            pltpu.VMEM((C, D), jnp.float32),
            pltpu.VMEM((ROWS_PER_W,), jnp.float32),
            pltpu.VMEM((ROWS_PER_W,), jnp.float32),
            pltpu.SemaphoreType.DMA((2,)),
            pltpu.SemaphoreType.DMA((2,)),
        ],
    )
    def run(h_hbm, s0_hbm, s1_hbm, out_hbm, ib0, ib1, ob0, ob1, s0b, s1b,
            isem, osem):
        ibufs = (ib0, ib1)
        obufs = (ob0, ob1)
        wid = lax.axis_index("s") * NC + lax.axis_index("c")
        base = wid * ROWS_PER_W
        pltpu.sync_copy(s0_hbm.at[pl.ds(base, ROWS_PER_W)], s0b)
        pltpu.sync_copy(s1_hbm.at[pl.ds(base, ROWS_PER_W)], s1b)

        def start_in(g, b):
            pltpu.async_copy(h_hbm.at[pl.ds(base + g * C, C)], ibufs[b],
                             isem.at[b])

        def wait_in(b):
            pltpu.make_async_copy(h_hbm.at[pl.ds(0, C)], ibufs[b],
                                  isem.at[b]).wait()

        def start_out(g, b):
            pltpu.async_copy(obufs[b], out_hbm.at[pl.ds(base + g * C, C)],
                             osem.at[b])

        def wait_out(b):
            pltpu.make_async_copy(obufs[b], out_hbm.at[pl.ds(0, C)],
                                  osem.at[b]).wait()

        def compute_chunk(svec, b):
            ib, ob = ibufs[b], obufs[b]
            for r in range(C):
                ssum = svec[C * b + r]

                @plsc.parallel_loop(0, D, step=16, unroll=8)
                def _vec(j):
                    ob[r, pl.ds(j, 16)] = ib[r, pl.ds(j, 16)] * ssum

        start_in(0, 0)
        start_in(1, 1)

        @pl.loop(0, NGRP)
        def _grp(k):
            svec = s0b[pl.ds(k * 16, 16)] + s1b[pl.ds(k * 16, 16)]
            for b in range(2):
                g = 2 * k + b
                wait_in(b)

                @pl.when(k >= 1)
                def _():
                    wait_out(b)

                compute_chunk(svec, b)
                start_out(g, b)

                @pl.when(k < NGRP - 1)
                def _():
                    start_in(g + 2, b)

        wait_out(0)
        wait_out(1)

    return run(hidden_states, s0, s1)


# speedup vs baseline: 1.5634x; 1.0003x over previous
import functools

import jax
import jax.numpy as jnp
from jax import lax
from jax.experimental import pallas as pl
from jax.experimental.pallas import tpu as pltpu
from jax.experimental.pallas import tpu_sc as plsc

T, D = 32768, 2048
NC, NS = 2, 16
NW = NC * NS
ROWS_PER_W = T // NW
C = 8
NCH = ROWS_PER_W // C
NBI = 4
NBO = 2
NGRP = NCH // NBI


def kernel(hidden_states, expert_scales):
    s0 = expert_scales[:, 0]
    s1 = expert_scales[:, 1]
    mesh = plsc.VectorSubcoreMesh(core_axis_name="c", subcore_axis_name="s")

    @functools.partial(
        pl.kernel,
        out_type=jax.ShapeDtypeStruct((T, D), jnp.float32),
        mesh=mesh,
        scratch_types=[
            pltpu.VMEM((C, D), jnp.float32),
            pltpu.VMEM((C, D), jnp.float32),
            pltpu.VMEM((C, D), jnp.float32),
            pltpu.VMEM((C, D), jnp.float32),
            pltpu.VMEM((C, D), jnp.float32),
            pltpu.VMEM((C, D), jnp.float32),
            pltpu.VMEM((ROWS_PER_W,), jnp.float32),
            pltpu.VMEM((ROWS_PER_W,), jnp.float32),
            pltpu.SemaphoreType.DMA((NBI,)),
            pltpu.SemaphoreType.DMA((NBO,)),
        ],
    )
    def run(h_hbm, s0_hbm, s1_hbm, out_hbm, ib0, ib1, ib2, ib3, ob0, ob1,
            s0b, s1b, isem, osem):
        ibufs = (ib0, ib1, ib2, ib3)
        obufs = (ob0, ob1)
        wid = lax.axis_index("s") * NC + lax.axis_index("c")
        base = wid * ROWS_PER_W
        pltpu.sync_copy(s0_hbm.at[pl.ds(base, ROWS_PER_W)], s0b)
        pltpu.sync_copy(s1_hbm.at[pl.ds(base, ROWS_PER_W)], s1b)

        def start_in(g, bi):
            pltpu.async_copy(h_hbm.at[pl.ds(base + g * C, C)], ibufs[bi],
                             isem.at[bi])

        def wait_in(bi):
            pltpu.make_async_copy(h_hbm.at[pl.ds(0, C)], ibufs[bi],
                                  isem.at[bi]).wait()

        def start_out(g, bo):
            pltpu.async_copy(obufs[bo], out_hbm.at[pl.ds(base + g * C, C)],
                             osem.at[bo])

        def wait_out(bo):
            pltpu.make_async_copy(obufs[bo], out_hbm.at[pl.ds(0, C)],
                                  osem.at[bo]).wait()

        def compute_chunk(svec, bi, bo, half):
            ib, ob = ibufs[bi], obufs[bo]
            for r in range(C):
                ssum = svec[C * half + r]

                @plsc.parallel_loop(0, D, step=16, unroll=8)
                def _vec(j):
                    ob[r, pl.ds(j, 16)] = ib[r, pl.ds(j, 16)] * ssum

        for b in range(NBI):
            start_in(b, b)

        @pl.loop(0, NGRP)
        def _grp(k):
            sva = s0b[pl.ds(k * 32, 16)] + s1b[pl.ds(k * 32, 16)]
            svb = s0b[pl.ds(k * 32 + 16, 16)] + s1b[pl.ds(k * 32 + 16, 16)]
            for b in range(NBI):
                g = NBI * k + b
                bo = b % NBO
                wait_in(b)

                @pl.when((k >= 1) | (b >= NBO))
                def _():
                    wait_out(bo)

                svec = sva if b < 2 else svb
                compute_chunk(svec, b, bo, b % 2)
                start_out(g, bo)

                @pl.when(k < NGRP - 1)
                def _():
                    start_in(g + NBI, b)

        wait_out(0)
        wait_out(1)

    return run(hidden_states, s0, s1)


# speedup vs baseline: 1.5656x; 1.0014x over previous
import functools

import jax
import jax.numpy as jnp
from jax import lax
from jax.experimental import pallas as pl
from jax.experimental.pallas import tpu as pltpu
from jax.experimental.pallas import tpu_sc as plsc

T, D = 32768, 2048
NC, NS = 2, 16
NW = NC * NS
ROWS_PER_W = T // NW
C = 8
NCH = ROWS_PER_W // C
NGRP = NCH // 2


def kernel(hidden_states, expert_scales):
    s0 = expert_scales[:, 0]
    s1 = expert_scales[:, 1]
    mesh = plsc.VectorSubcoreMesh(core_axis_name="c", subcore_axis_name="s")

    @functools.partial(
        pl.kernel,
        out_type=jax.ShapeDtypeStruct((T, D), jnp.float32),
        mesh=mesh,
        scratch_types=[
            pltpu.VMEM((C, D), jnp.float32),
            pltpu.VMEM((C, D), jnp.float32),
            pltpu.VMEM((C, D), jnp.float32),
            pltpu.VMEM((C, D), jnp.float32),
            pltpu.VMEM((ROWS_PER_W,), jnp.float32),
            pltpu.VMEM((ROWS_PER_W,), jnp.float32),
            pltpu.SemaphoreType.DMA((2,)),
            pltpu.SemaphoreType.DMA((2,)),
        ],
    )
    def run(h_hbm, s0_hbm, s1_hbm, out_hbm, ib0, ib1, ob0, ob1, s0b, s1b,
            isem, osem):
        ibufs = (ib0, ib1)
        obufs = (ob0, ob1)
        wid = lax.axis_index("s") * NC + lax.axis_index("c")
        base = wid * ROWS_PER_W
        pltpu.sync_copy(s0_hbm.at[pl.ds(base, ROWS_PER_W)], s0b)
        pltpu.sync_copy(s1_hbm.at[pl.ds(base, ROWS_PER_W)], s1b)

        def start_in(g, b):
            pltpu.async_copy(h_hbm.at[pl.ds(base + g * C, C)], ibufs[b],
                             isem.at[b])

        def wait_in(b):
            pltpu.make_async_copy(h_hbm.at[pl.ds(0, C)], ibufs[b],
                                  isem.at[b]).wait()

        def start_out(g, b):
            pltpu.async_copy(obufs[b], out_hbm.at[pl.ds(base + g * C, C)],
                             osem.at[b])

        def wait_out(b):
            pltpu.make_async_copy(obufs[b], out_hbm.at[pl.ds(0, C)],
                                  osem.at[b]).wait()

        def compute_chunk(svec, b):
            ib, ob = ibufs[b], obufs[b]
            ssums = [svec[C * b + r] for r in range(C)]

            @plsc.parallel_loop(0, D, step=16, unroll=2)
            def _vec(j):
                for r in range(C):
                    ob[r, pl.ds(j, 16)] = ib[r, pl.ds(j, 16)] * ssums[r]

        start_in(0, 0)
        start_in(1, 1)

        @pl.loop(0, NGRP)
        def _grp(k):
            svec = s0b[pl.ds(k * 16, 16)] + s1b[pl.ds(k * 16, 16)]
            for b in range(2):
                g = 2 * k + b
                wait_in(b)

                @pl.when(k >= 1)
                def _():
                    wait_out(b)

                compute_chunk(svec, b)
                start_out(g, b)

                @pl.when(k < NGRP - 1)
                def _():
                    start_in(g + 2, b)

        wait_out(0)
        wait_out(1)

    return run(hidden_states, s0, s1)


# speedup vs baseline: 1.5715x; 1.0038x over previous
import functools

import jax
import jax.numpy as jnp
from jax import lax
from jax.experimental import pallas as pl
from jax.experimental.pallas import tpu as pltpu
from jax.experimental.pallas import tpu_sc as plsc

T, D = 32768, 2048
NC, NS = 2, 16
NW = NC * NS
ROWS_PER_W = T // NW
C = 8
NCH = ROWS_PER_W // C
NGRP = NCH // 2


def kernel(hidden_states, expert_scales):
    s0 = expert_scales[:, 0]
    s1 = expert_scales[:, 1]
    mesh = plsc.VectorSubcoreMesh(core_axis_name="c", subcore_axis_name="s")

    @functools.partial(
        pl.kernel,
        out_type=jax.ShapeDtypeStruct((T, D), jnp.float32),
        mesh=mesh,
        scratch_types=[
            pltpu.VMEM((C, D), jnp.float32),
            pltpu.VMEM((C, D), jnp.float32),
            pltpu.VMEM((C, D), jnp.float32),
            pltpu.VMEM((C, D), jnp.float32),
            pltpu.VMEM((ROWS_PER_W,), jnp.float32),
            pltpu.VMEM((ROWS_PER_W,), jnp.float32),
            pltpu.SemaphoreType.DMA((2,)),
            pltpu.SemaphoreType.DMA((2,)),
            pltpu.SemaphoreType.DMA,
        ],
    )
    def run(h_hbm, s0_hbm, s1_hbm, out_hbm, ib0, ib1, ob0, ob1, s0b, s1b,
            isem, osem, ssem):
        ibufs = (ib0, ib1)
        obufs = (ob0, ob1)
        wid = lax.axis_index("s") * NC + lax.axis_index("c")
        base = wid * ROWS_PER_W

        def start_in(g, b):
            pltpu.async_copy(h_hbm.at[pl.ds(base + g * C, C)], ibufs[b],
                             isem.at[b])

        def wait_in(b):
            pltpu.make_async_copy(h_hbm.at[pl.ds(0, C)], ibufs[b],
                                  isem.at[b]).wait()

        def start_out(g, b):
            pltpu.async_copy(obufs[b], out_hbm.at[pl.ds(base + g * C, C)],
                             osem.at[b])

        def wait_out(b):
            pltpu.make_async_copy(obufs[b], out_hbm.at[pl.ds(0, C)],
                                  osem.at[b]).wait()

        def compute_chunk(svec, b):
            ib, ob = ibufs[b], obufs[b]
            ssums = [svec[C * b + r] for r in range(C)]

            @plsc.parallel_loop(0, D, step=16, unroll=2)
            def _vec(j):
                for r in range(C):
                    ob[r, pl.ds(j, 16)] = ib[r, pl.ds(j, 16)] * ssums[r]

        start_in(0, 0)
        start_in(1, 1)
        c0 = pltpu.async_copy(s0_hbm.at[pl.ds(base, ROWS_PER_W)], s0b, ssem)
        c1 = pltpu.async_copy(s1_hbm.at[pl.ds(base, ROWS_PER_W)], s1b, ssem)
        c0.wait()
        c1.wait()

        @pl.loop(0, NGRP)
        def _grp(k):
            svec = s0b[pl.ds(k * 16, 16)] + s1b[pl.ds(k * 16, 16)]
            for b in range(2):
                g = 2 * k + b
                wait_in(b)

                @pl.when(k >= 1)
                def _():
                    wait_out(b)

                compute_chunk(svec, b)
                start_out(g, b)

                @pl.when(k < NGRP - 1)
                def _():
                    start_in(g + 2, b)

        wait_out(0)
        wait_out(1)

    return run(hidden_states, s0, s1)
